# baseline (speedup 1.0000x reference)
import functools

import jax
import jax.numpy as jnp
from jax import lax
from jax.experimental import pallas as pl
from jax.experimental.pallas import tpu as pltpu
from jax.experimental.pallas import tpu_sc as plsc

B, T, N, E = 4, 4, 10000, 160000
D_IN, H_GNN, H_RNN, H_FC = 128, 128, 128, 128
D_S, D_T, OUT_S, OUT_C = 64, 64, 10, 10
G = B * T
NC, NS = 2, 16
NW = NC * NS
EPT = E // NW
CH = 100
CPT = EPT // CH
U = 10
RPT = N // NS


def _sc_agg_body(with_deg, x_ref, src_ref, dst_ref, *rest):
    if with_deg:
        (y0_ref, y1_ref, d0_ref, d1_ref, src_v, dst_v, rows0, rows1,
         y_sh, gsem0, gsem1, ssem0, ssem1) = rest
    else:
        (y0_ref, y1_ref, src_v, dst_v, rows0, rows1, y_sh,
         gsem0, gsem1, ssem0, ssem1) = rest
    c = lax.axis_index("c")
    s = lax.axis_index("s")
    w = c * NS + s

    def _fill(buf, val):
        def _fb(t, carry):
            r = t // 8
            j = t % 8
            buf[r, pl.ds(j * 16, 16)] = jnp.full((16,), val, jnp.float32)
            return carry
        lax.fori_loop(0, CH * 8, _fb, 0)

    def _zero_own_rows():
        _fill(rows0, 0.0)
        for j in range(6):
            pltpu.sync_copy(rows0, y_sh.at[pl.ds(s * RPT + j * CH, CH)])
        pltpu.sync_copy(rows0.at[pl.ds(0, 25)],
                        y_sh.at[pl.ds(s * RPT + 600, 25)])

    def _scatter(buf, k):
        pltpu.sync_copy(buf, y_sh.at[dst_v.at[k]], add=True)

    pltpu.sync_copy(dst_ref.at[w], dst_v)

    if with_deg:
        _zero_own_rows()
        _fill(rows0, 1.0)
        plsc.subcore_barrier()

        def _dchunk(k, carry):
            pltpu.sync_copy(rows0, y_sh.at[dst_v.at[k]], add=True)
            return carry
        lax.fori_loop(0, CPT, _dchunk, 0)
        plsc.subcore_barrier()

        @pl.when(c == 0)
        def _():
            pltpu.sync_copy(y_sh.at[pl.ds(s * RPT, RPT)], d0_ref.at[s])

        @pl.when(c == 1)
        def _():
            pltpu.sync_copy(y_sh.at[pl.ds(s * RPT, RPT)], d1_ref.at[s])

    def _inst(i, carry):
        _zero_own_rows()
        pltpu.sync_copy(src_ref.at[i * NW + w], src_v)
        plsc.subcore_barrier()

        bufs = (rows0, rows1)
        gsems = (gsem0, gsem1)
        ssems = (ssem0, ssem1)

        def _block(p, carry2):
            base = p * U
            gd = [None] * U
            sd = [None] * U
            gd[0] = pltpu.async_copy(
                x_ref.at[src_v.at[base]], bufs[0], gsems[0])
            for j in range(U):
                gd[j].wait()
                sd[j] = pltpu.async_copy(
                    bufs[j % 2], y_sh.at[dst_v.at[base + j]],
                    ssems[j % 2], add=True)
                if j + 1 < U:
                    if j >= 1:
                        sd[j - 1].wait()
                    gd[j + 1] = pltpu.async_copy(
                        x_ref.at[src_v.at[base + j + 1]],
                        bufs[(j + 1) % 2], gsems[(j + 1) % 2])
            sd[U - 2].wait()
            sd[U - 1].wait()
            return carry2
        lax.fori_loop(0, CPT // U, _block, 0)
        plsc.subcore_barrier()

        @pl.when(c == 0)
        def _():
            pltpu.sync_copy(y_sh.at[pl.ds(s * RPT, RPT)], y0_ref.at[i, s])

        @pl.when(c == 1)
        def _():
            pltpu.sync_copy(y_sh.at[pl.ds(s * RPT, RPT)], y1_ref.at[i, s])
        plsc.subcore_barrier()
        return carry
    lax.fori_loop(0, G, _inst, 0)


def _make_sc_agg(with_deg):
    yshape = jax.ShapeDtypeStruct((G, NS, RPT, H_GNN), jnp.float32)
    dshape = jax.ShapeDtypeStruct((NS, RPT, H_GNN), jnp.float32)
    out_type = (yshape, yshape, dshape, dshape) if with_deg else (yshape, yshape)
    scratch = [
        pltpu.VMEM((CPT, CH), jnp.int32),
        pltpu.VMEM((CPT, CH), jnp.int32),
        pltpu.VMEM((CH, H_GNN), jnp.float32),
        pltpu.VMEM((CH, H_GNN), jnp.float32),
    ]
    scratch += [pltpu.VMEM_SHARED((N, H_GNN), jnp.float32)]
    scratch += [pltpu.SemaphoreType.DMA, pltpu.SemaphoreType.DMA,
                pltpu.SemaphoreType.DMA, pltpu.SemaphoreType.DMA]
    mesh = plsc.VectorSubcoreMesh(core_axis_name="c", subcore_axis_name="s")
    return pl.kernel(
        functools.partial(_sc_agg_body, with_deg),
        out_type=out_type,
        mesh=mesh,
        scratch_types=scratch,
    )


BN = 2000
NB = N // BN


def _tc_layer1_body(y0, y1, d0, d1, wt, b, z_out):
    y = y0[0] + y1[0]
    deg = d0[:, 0:1] + d1[:, 0:1]
    m = y * (1.0 / jnp.maximum(deg, 1.0))
    z = jnp.dot(m, wt[...], preferred_element_type=jnp.float32) + b[...]
    z_out[0] = jnp.maximum(z, 0.0)


def _tc_layer2_body(y0, y1, d0, d1, wt, b, r_out):
    nb = pl.program_id(1)
    y = y0[0] + y1[0]
    deg = d0[:, 0:1] + d1[:, 0:1]
    m = y * (1.0 / jnp.maximum(deg, 1.0))
    z = jnp.dot(m, wt[...], preferred_element_type=jnp.float32) + b[...]
    z = jnp.maximum(z, 0.0)
    part = (jnp.sum(z, axis=0, keepdims=True) * (1.0 / N))[None]

    @pl.when(nb == 0)
    def _():
        r_out[...] = part

    @pl.when(nb != 0)
    def _():
        r_out[...] = r_out[...] + part


def _tc_layer(emit_z):
    in_specs = [
        pl.BlockSpec((1, BN, H_GNN), lambda i, nb: (i, nb, 0)),
        pl.BlockSpec((1, BN, H_GNN), lambda i, nb: (i, nb, 0)),
        pl.BlockSpec((BN, H_GNN), lambda i, nb: (nb, 0)),
        pl.BlockSpec((BN, H_GNN), lambda i, nb: (nb, 0)),
        pl.BlockSpec((H_GNN, H_GNN), lambda i, nb: (0, 0)),
        pl.BlockSpec((1, H_GNN), lambda i, nb: (0, 0)),
    ]
    if emit_z:
        return pl.pallas_call(
            _tc_layer1_body,
            grid=(G, NB),
            in_specs=in_specs,
            out_specs=pl.BlockSpec((1, BN, H_GNN), lambda i, nb: (i, nb, 0)),
            out_shape=jax.ShapeDtypeStruct((G, N, H_GNN), jnp.float32),
        )
    return pl.pallas_call(
        _tc_layer2_body,
        grid=(G, NB),
        in_specs=in_specs,
        out_specs=pl.BlockSpec((1, 1, H_GNN), lambda i, nb: (i, 0, 0)),
        out_shape=jax.ShapeDtypeStruct((G, 1, H_GNN), jnp.float32),
    )


def _gru_heads_body(g_ref, s_ref, t_ref,
                    wihg, whhg, bihg, bhhg,
                    wihs, whhs, bihs, bhhs,
                    wiht, whht, biht, bhht,
                    wfc, bfc, wst, bst, wca, bca,
                    stim_ref, cause_ref):
    H = H_FC

    def gru(seq, wih, whh, bih, bhh):
        h = jnp.zeros((B, H), jnp.float32)
        hs = []
        for t in range(T):
            x = seq[:, t, :]
            gi = jnp.dot(x, wih[...], preferred_element_type=jnp.float32) + bih[...]
            gh = jnp.dot(h, whh[...], preferred_element_type=jnp.float32) + bhh[...]
            r = jax.nn.sigmoid(gi[:, 0:H] + gh[:, 0:H])
            z = jax.nn.sigmoid(gi[:, H:2 * H] + gh[:, H:2 * H])
            n = jnp.tanh(gi[:, 2 * H:3 * H] + r * gh[:, 2 * H:3 * H])
            h = (1.0 - z) * n + z * h
            hs.append(h)
        return hs

    hg = gru(g_ref[...], wihg, whhg, bihg, bhhg)
    hs_ = gru(s_ref[...], wihs, whhs, bihs, bhhs)
    ht = gru(t_ref[...], wiht, whht, biht, bhht)
    for t in range(T):
        cat = jnp.concatenate([hg[t], hs_[t], ht[t]], axis=1)
        hO = jnp.dot(cat, wfc[...], preferred_element_type=jnp.float32) + bfc[...]
        hO = jnp.maximum(hO, 0.0)
        stim_ref[:, t, :] = jnp.dot(hO, wst[...], preferred_element_type=jnp.float32) + bst[...]
        cause_ref[:, t, :] = jnp.dot(hO, wca[...], preferred_element_type=jnp.float32) + bca[...]


_gru_heads = pl.pallas_call(
    _gru_heads_body,
    out_shape=(jax.ShapeDtypeStruct((B, T, OUT_S), jnp.float32),
               jax.ShapeDtypeStruct((B, T, OUT_C), jnp.float32)),
)


def kernel(node_feats, edge_index, bSensor, bTarget, bArea,
           W_gnn1, b_gnn1, W_gnn3, b_gnn3,
           W_ih_G, W_hh_G, b_ih_G, b_hh_G,
           W_ih_S, W_hh_S, b_ih_S, b_hh_S,
           W_ih_T, W_hh_T, b_ih_T, b_hh_T,
           W_fc1, b_fc1, W_stim, b_stim, W_cause, b_cause):
    src = edge_index[0]
    dst = edge_index[1]
    offs = (jnp.arange(G, dtype=jnp.int32) * N)[:, None]
    src_rows = (src[None, :] + offs).reshape(G * NW, CPT, CH)
    dst_rows = dst.reshape(NW, CPT, CH)

    x1 = node_feats.reshape(G * N, D_IN)
    y0a, y1a, deg0, deg1 = _make_sc_agg(True)(x1, src_rows, dst_rows)
    deg0 = deg0.reshape(N, H_GNN)
    deg1 = deg1.reshape(N, H_GNN)
    z1 = _tc_layer(True)(y0a.reshape(G, N, H_GNN), y1a.reshape(G, N, H_GNN),
                         deg0, deg1, W_gnn1.T, b_gnn1.reshape(1, -1))
    y0b, y1b = _make_sc_agg(False)(z1.reshape(G * N, H_GNN), src_rows, dst_rows)
    r = _tc_layer(False)(y0b.reshape(G, N, H_GNN), y1b.reshape(G, N, H_GNN),
                         deg0, deg1, W_gnn3.T, b_gnn3.reshape(1, -1))

    stim4, cause4 = _gru_heads(
        r.reshape(B, T, H_RNN), bSensor, bTarget,
        W_ih_G.T, W_hh_G.T, b_ih_G.reshape(1, -1), b_hh_G.reshape(1, -1),
        W_ih_S.T, W_hh_S.T, b_ih_S.reshape(1, -1), b_hh_S.reshape(1, -1),
        W_ih_T.T, W_hh_T.T, b_ih_T.reshape(1, -1), b_hh_T.reshape(1, -1),
        W_fc1.T, b_fc1.reshape(1, -1),
        W_stim.T, b_stim.reshape(1, -1),
        W_cause.T, b_cause.reshape(1, -1))
    return (stim4.reshape(B * T, OUT_S), cause4.reshape(B * T, OUT_C))


# speedup vs baseline: 1.0479x; 1.0479x over previous
import functools

import jax
import jax.numpy as jnp
from jax import lax
from jax.experimental import pallas as pl
from jax.experimental.pallas import tpu as pltpu
from jax.experimental.pallas import tpu_sc as plsc

B, T, N, E = 4, 4, 10000, 160000
D_IN, H_GNN, H_RNN, H_FC = 128, 128, 128, 128
D_S, D_T, OUT_S, OUT_C = 64, 64, 10, 10
G = B * T
NC, NS = 2, 16
NW = NC * NS
EPT = E // NW
CH = 125
CPT = EPT // CH
U = 8
RPT = N // NS


def _sc_agg_body(with_deg, x_ref, src_ref, dst_ref, *rest):
    if with_deg:
        (y0_ref, y1_ref, d0_ref, d1_ref, src_v, dst_v, rows0, rows1,
         y_sh, gsem0, gsem1, ssem0, ssem1) = rest
    else:
        (y0_ref, y1_ref, src_v, dst_v, rows0, rows1, y_sh,
         gsem0, gsem1, ssem0, ssem1) = rest
    c = lax.axis_index("c")
    s = lax.axis_index("s")
    w = c * NS + s

    def _fill(buf, val):
        def _fb(t, carry):
            r = t // 8
            j = t % 8
            buf[r, pl.ds(j * 16, 16)] = jnp.full((16,), val, jnp.float32)
            return carry
        lax.fori_loop(0, CH * 8, _fb, 0)

    def _zero_own_rows():
        _fill(rows0, 0.0)
        for j in range(RPT // CH):
            pltpu.sync_copy(rows0, y_sh.at[pl.ds(s * RPT + j * CH, CH)])

    def _scatter(buf, k):
        pltpu.sync_copy(buf, y_sh.at[dst_v.at[k]], add=True)

    pltpu.sync_copy(dst_ref.at[w], dst_v)

    if with_deg:
        _zero_own_rows()
        _fill(rows0, 1.0)
        plsc.subcore_barrier()

        def _dchunk(k, carry):
            pltpu.sync_copy(rows0, y_sh.at[dst_v.at[k]], add=True)
            return carry
        lax.fori_loop(0, CPT, _dchunk, 0)
        plsc.subcore_barrier()

        @pl.when(c == 0)
        def _():
            pltpu.sync_copy(y_sh.at[pl.ds(s * RPT, RPT)], d0_ref.at[s])

        @pl.when(c == 1)
        def _():
            pltpu.sync_copy(y_sh.at[pl.ds(s * RPT, RPT)], d1_ref.at[s])

    def _inst(i, carry):
        _zero_own_rows()
        pltpu.sync_copy(src_ref.at[i * NW + w], src_v)
        plsc.subcore_barrier()

        bufs = (rows0, rows1)
        gsems = (gsem0, gsem1)
        ssems = (ssem0, ssem1)

        def _block(p, carry2):
            base = p * U
            gd = [None] * U
            sd = [None] * U
            gd[0] = pltpu.async_copy(
                x_ref.at[src_v.at[base]], bufs[0], gsems[0])
            for j in range(U):
                gd[j].wait()
                sd[j] = pltpu.async_copy(
                    bufs[j % 2], y_sh.at[dst_v.at[base + j]],
                    ssems[j % 2], add=True)
                if j + 1 < U:
                    if j >= 1:
                        sd[j - 1].wait()
                    gd[j + 1] = pltpu.async_copy(
                        x_ref.at[src_v.at[base + j + 1]],
                        bufs[(j + 1) % 2], gsems[(j + 1) % 2])
            sd[U - 2].wait()
            sd[U - 1].wait()
            return carry2
        lax.fori_loop(0, CPT // U, _block, 0)
        plsc.subcore_barrier()

        @pl.when(c == 0)
        def _():
            pltpu.sync_copy(y_sh.at[pl.ds(s * RPT, RPT)], y0_ref.at[i, s])

        @pl.when(c == 1)
        def _():
            pltpu.sync_copy(y_sh.at[pl.ds(s * RPT, RPT)], y1_ref.at[i, s])
        plsc.subcore_barrier()
        return carry
    lax.fori_loop(0, G, _inst, 0)


def _make_sc_agg(with_deg):
    yshape = jax.ShapeDtypeStruct((G, NS, RPT, H_GNN), jnp.float32)
    dshape = jax.ShapeDtypeStruct((NS, RPT, H_GNN), jnp.float32)
    out_type = (yshape, yshape, dshape, dshape) if with_deg else (yshape, yshape)
    scratch = [
        pltpu.VMEM((CPT, CH), jnp.int32),
        pltpu.VMEM((CPT, CH), jnp.int32),
        pltpu.VMEM((CH, H_GNN), jnp.float32),
        pltpu.VMEM((CH, H_GNN), jnp.float32),
    ]
    scratch += [pltpu.VMEM_SHARED((N, H_GNN), jnp.float32)]
    scratch += [pltpu.SemaphoreType.DMA, pltpu.SemaphoreType.DMA,
                pltpu.SemaphoreType.DMA, pltpu.SemaphoreType.DMA]
    mesh = plsc.VectorSubcoreMesh(core_axis_name="c", subcore_axis_name="s")
    return pl.kernel(
        functools.partial(_sc_agg_body, with_deg),
        out_type=out_type,
        mesh=mesh,
        scratch_types=scratch,
    )


BN = 2000
NB = N // BN


def _tc_layer1_body(y0, y1, d0, d1, wt, b, z_out):
    y = y0[0] + y1[0]
    deg = d0[:, 0:1] + d1[:, 0:1]
    m = y * (1.0 / jnp.maximum(deg, 1.0))
    z = jnp.dot(m, wt[...], preferred_element_type=jnp.float32) + b[...]
    z_out[0] = jnp.maximum(z, 0.0)


def _tc_layer2_body(y0, y1, d0, d1, wt, b, r_out):
    nb = pl.program_id(1)
    y = y0[0] + y1[0]
    deg = d0[:, 0:1] + d1[:, 0:1]
    m = y * (1.0 / jnp.maximum(deg, 1.0))
    z = jnp.dot(m, wt[...], preferred_element_type=jnp.float32) + b[...]
    z = jnp.maximum(z, 0.0)
    part = (jnp.sum(z, axis=0, keepdims=True) * (1.0 / N))[None]

    @pl.when(nb == 0)
    def _():
        r_out[...] = part

    @pl.when(nb != 0)
    def _():
        r_out[...] = r_out[...] + part


def _tc_layer(emit_z):
    in_specs = [
        pl.BlockSpec((1, BN, H_GNN), lambda i, nb: (i, nb, 0)),
        pl.BlockSpec((1, BN, H_GNN), lambda i, nb: (i, nb, 0)),
        pl.BlockSpec((BN, H_GNN), lambda i, nb: (nb, 0)),
        pl.BlockSpec((BN, H_GNN), lambda i, nb: (nb, 0)),
        pl.BlockSpec((H_GNN, H_GNN), lambda i, nb: (0, 0)),
        pl.BlockSpec((1, H_GNN), lambda i, nb: (0, 0)),
    ]
    if emit_z:
        return pl.pallas_call(
            _tc_layer1_body,
            grid=(G, NB),
            in_specs=in_specs,
            out_specs=pl.BlockSpec((1, BN, H_GNN), lambda i, nb: (i, nb, 0)),
            out_shape=jax.ShapeDtypeStruct((G, N, H_GNN), jnp.float32),
        )
    return pl.pallas_call(
        _tc_layer2_body,
        grid=(G, NB),
        in_specs=in_specs,
        out_specs=pl.BlockSpec((1, 1, H_GNN), lambda i, nb: (i, 0, 0)),
        out_shape=jax.ShapeDtypeStruct((G, 1, H_GNN), jnp.float32),
    )


def _gru_heads_body(g_ref, s_ref, t_ref,
                    wihg, whhg, bihg, bhhg,
                    wihs, whhs, bihs, bhhs,
                    wiht, whht, biht, bhht,
                    wfc, bfc, wst, bst, wca, bca,
                    stim_ref, cause_ref):
    H = H_FC

    def gru(seq, wih, whh, bih, bhh):
        h = jnp.zeros((B, H), jnp.float32)
        hs = []
        for t in range(T):
            x = seq[:, t, :]
            gi = jnp.dot(x, wih[...], preferred_element_type=jnp.float32) + bih[...]
            gh = jnp.dot(h, whh[...], preferred_element_type=jnp.float32) + bhh[...]
            r = jax.nn.sigmoid(gi[:, 0:H] + gh[:, 0:H])
            z = jax.nn.sigmoid(gi[:, H:2 * H] + gh[:, H:2 * H])
            n = jnp.tanh(gi[:, 2 * H:3 * H] + r * gh[:, 2 * H:3 * H])
            h = (1.0 - z) * n + z * h
            hs.append(h)
        return hs

    hg = gru(g_ref[...], wihg, whhg, bihg, bhhg)
    hs_ = gru(s_ref[...], wihs, whhs, bihs, bhhs)
    ht = gru(t_ref[...], wiht, whht, biht, bhht)
    for t in range(T):
        cat = jnp.concatenate([hg[t], hs_[t], ht[t]], axis=1)
        hO = jnp.dot(cat, wfc[...], preferred_element_type=jnp.float32) + bfc[...]
        hO = jnp.maximum(hO, 0.0)
        stim_ref[:, t, :] = jnp.dot(hO, wst[...], preferred_element_type=jnp.float32) + bst[...]
        cause_ref[:, t, :] = jnp.dot(hO, wca[...], preferred_element_type=jnp.float32) + bca[...]


_gru_heads = pl.pallas_call(
    _gru_heads_body,
    out_shape=(jax.ShapeDtypeStruct((B, T, OUT_S), jnp.float32),
               jax.ShapeDtypeStruct((B, T, OUT_C), jnp.float32)),
)


def kernel(node_feats, edge_index, bSensor, bTarget, bArea,
           W_gnn1, b_gnn1, W_gnn3, b_gnn3,
           W_ih_G, W_hh_G, b_ih_G, b_hh_G,
           W_ih_S, W_hh_S, b_ih_S, b_hh_S,
           W_ih_T, W_hh_T, b_ih_T, b_hh_T,
           W_fc1, b_fc1, W_stim, b_stim, W_cause, b_cause):
    src = edge_index[0]
    dst = edge_index[1]
    offs = (jnp.arange(G, dtype=jnp.int32) * N)[:, None]
    src_rows = (src[None, :] + offs).reshape(G * NW, CPT, CH)
    dst_rows = dst.reshape(NW, CPT, CH)

    x1 = node_feats.reshape(G * N, D_IN)
    y0a, y1a, deg0, deg1 = _make_sc_agg(True)(x1, src_rows, dst_rows)
    deg0 = deg0.reshape(N, H_GNN)
    deg1 = deg1.reshape(N, H_GNN)
    z1 = _tc_layer(True)(y0a.reshape(G, N, H_GNN), y1a.reshape(G, N, H_GNN),
                         deg0, deg1, W_gnn1.T, b_gnn1.reshape(1, -1))
    y0b, y1b = _make_sc_agg(False)(z1.reshape(G * N, H_GNN), src_rows, dst_rows)
    r = _tc_layer(False)(y0b.reshape(G, N, H_GNN), y1b.reshape(G, N, H_GNN),
                         deg0, deg1, W_gnn3.T, b_gnn3.reshape(1, -1))

    stim4, cause4 = _gru_heads(
        r.reshape(B, T, H_RNN), bSensor, bTarget,
        W_ih_G.T, W_hh_G.T, b_ih_G.reshape(1, -1), b_hh_G.reshape(1, -1),
        W_ih_S.T, W_hh_S.T, b_ih_S.reshape(1, -1), b_hh_S.reshape(1, -1),
        W_ih_T.T, W_hh_T.T, b_ih_T.reshape(1, -1), b_hh_T.reshape(1, -1),
        W_fc1.T, b_fc1.reshape(1, -1),
        W_stim.T, b_stim.reshape(1, -1),
        W_cause.T, b_cause.reshape(1, -1))
    return (stim4.reshape(B * T, OUT_S), cause4.reshape(B * T, OUT_C))


# speedup vs baseline: 1.0777x; 1.0284x over previous
import functools

import jax
import jax.numpy as jnp
from jax import lax
from jax.experimental import pallas as pl
from jax.experimental.pallas import tpu as pltpu
from jax.experimental.pallas import tpu_sc as plsc

B, T, N, E = 4, 4, 10000, 160000
D_IN, H_GNN, H_RNN, H_FC = 128, 128, 128, 128
D_S, D_T, OUT_S, OUT_C = 64, 64, 10, 10
G = B * T
NC, NS = 2, 16
NW = NC * NS
EPT = E // NW
CH = 125
CPT = EPT // CH
U = 10
RPT = N // NS


def _sc_agg_body(with_deg, x_ref, src_ref, dst_ref, *rest):
    if with_deg:
        (y0_ref, y1_ref, d0_ref, d1_ref, src_v, dst_v, rows0, rows1,
         y_sh, gsem0, gsem1, ssem0, ssem1) = rest
    else:
        (y0_ref, y1_ref, src_v, dst_v, rows0, rows1, y_sh,
         gsem0, gsem1, ssem0, ssem1) = rest
    c = lax.axis_index("c")
    s = lax.axis_index("s")
    w = c * NS + s

    def _fill(buf, val):
        def _fb(t, carry):
            r = t // 8
            j = t % 8
            buf[r, pl.ds(j * 16, 16)] = jnp.full((16,), val, jnp.float32)
            return carry
        lax.fori_loop(0, CH * 8, _fb, 0)

    def _zero_own_rows():
        _fill(rows0, 0.0)
        for j in range(RPT // CH):
            pltpu.sync_copy(rows0, y_sh.at[pl.ds(s * RPT + j * CH, CH)])

    def _scatter(buf, k):
        pltpu.sync_copy(buf, y_sh.at[dst_v.at[k]], add=True)

    pltpu.sync_copy(dst_ref.at[w], dst_v)
    pltpu.sync_copy(src_ref.at[w], src_v)

    if with_deg:
        _zero_own_rows()
        _fill(rows0, 1.0)
        plsc.subcore_barrier()

        def _dchunk(k, carry):
            pltpu.sync_copy(rows0, y_sh.at[dst_v.at[k]], add=True)
            return carry
        lax.fori_loop(0, CPT, _dchunk, 0)
        plsc.subcore_barrier()

        @pl.when(c == 0)
        def _():
            pltpu.sync_copy(y_sh.at[pl.ds(s * RPT, RPT)], d0_ref.at[s])

        @pl.when(c == 1)
        def _():
            pltpu.sync_copy(y_sh.at[pl.ds(s * RPT, RPT)], d1_ref.at[s])

    def _inst(i, carry):
        _zero_own_rows()
        plsc.subcore_barrier()

        bufs = (rows0, rows1)
        gsems = (gsem0, gsem1)
        ssems = (ssem0, ssem1)

        def _block(p, carry2):
            base = p * U
            gd = [None] * U
            sd = [None] * U
            gd[0] = pltpu.async_copy(
                x_ref.at[i].at[src_v.at[base]], bufs[0], gsems[0])
            for j in range(U):
                gd[j].wait()
                sd[j] = pltpu.async_copy(
                    bufs[j % 2], y_sh.at[dst_v.at[base + j]],
                    ssems[j % 2], add=True)
                if j + 1 < U:
                    if j >= 1:
                        sd[j - 1].wait()
                    gd[j + 1] = pltpu.async_copy(
                        x_ref.at[i].at[src_v.at[base + j + 1]],
                        bufs[(j + 1) % 2], gsems[(j + 1) % 2])
            sd[U - 2].wait()
            sd[U - 1].wait()
            return carry2
        lax.fori_loop(0, CPT // U, _block, 0)
        plsc.subcore_barrier()

        @pl.when(c == 0)
        def _():
            pltpu.sync_copy(y_sh.at[pl.ds(s * RPT, RPT)], y0_ref.at[i, s])

        @pl.when(c == 1)
        def _():
            pltpu.sync_copy(y_sh.at[pl.ds(s * RPT, RPT)], y1_ref.at[i, s])
        plsc.subcore_barrier()
        return carry
    lax.fori_loop(0, G, _inst, 0)


def _make_sc_agg(with_deg):
    yshape = jax.ShapeDtypeStruct((G, NS, RPT, H_GNN), jnp.float32)
    dshape = jax.ShapeDtypeStruct((NS, RPT, H_GNN), jnp.float32)
    out_type = (yshape, yshape, dshape, dshape) if with_deg else (yshape, yshape)
    scratch = [
        pltpu.VMEM((CPT, CH), jnp.int32),
        pltpu.VMEM((CPT, CH), jnp.int32),
        pltpu.VMEM((CH, H_GNN), jnp.float32),
        pltpu.VMEM((CH, H_GNN), jnp.float32),
    ]
    scratch += [pltpu.VMEM_SHARED((N, H_GNN), jnp.float32)]
    scratch += [pltpu.SemaphoreType.DMA, pltpu.SemaphoreType.DMA,
                pltpu.SemaphoreType.DMA, pltpu.SemaphoreType.DMA]
    mesh = plsc.VectorSubcoreMesh(core_axis_name="c", subcore_axis_name="s")
    return pl.kernel(
        functools.partial(_sc_agg_body, with_deg),
        out_type=out_type,
        mesh=mesh,
        scratch_types=scratch,
    )


BN = 2000
NB = N // BN


def _tc_layer1_body(y0, y1, d0, d1, wt, b, z_out):
    y = y0[0] + y1[0]
    deg = d0[:, 0:1] + d1[:, 0:1]
    m = y * (1.0 / jnp.maximum(deg, 1.0))
    z = jnp.dot(m, wt[...], preferred_element_type=jnp.float32) + b[...]
    z_out[0] = jnp.maximum(z, 0.0)


def _tc_layer2_body(y0, y1, d0, d1, wt, b, r_out):
    y = y0[0] + y1[0]
    deg = d0[:, 0:1] + d1[:, 0:1]
    m = y * (1.0 / jnp.maximum(deg, 1.0))
    z = jnp.dot(m, wt[...], preferred_element_type=jnp.float32) + b[...]
    z = jnp.maximum(z, 0.0)
    r_out[...] = (jnp.sum(z, axis=0, keepdims=True) * (1.0 / N))[None, None]


def _tc_layer(emit_z):
    in_specs = [
        pl.BlockSpec((1, BN, H_GNN), lambda nb, i: (i, nb, 0)),
        pl.BlockSpec((1, BN, H_GNN), lambda nb, i: (i, nb, 0)),
        pl.BlockSpec((BN, H_GNN), lambda nb, i: (nb, 0)),
        pl.BlockSpec((BN, H_GNN), lambda nb, i: (nb, 0)),
        pl.BlockSpec((H_GNN, H_GNN), lambda nb, i: (0, 0)),
        pl.BlockSpec((1, H_GNN), lambda nb, i: (0, 0)),
    ]
    if emit_z:
        return pl.pallas_call(
            _tc_layer1_body,
            grid=(NB, G),
            in_specs=in_specs,
            out_specs=pl.BlockSpec((1, BN, H_GNN), lambda nb, i: (i, nb, 0)),
            out_shape=jax.ShapeDtypeStruct((G, N, H_GNN), jnp.float32),
        )
    return pl.pallas_call(
        _tc_layer2_body,
        grid=(NB, G),
        in_specs=in_specs,
        out_specs=pl.BlockSpec((1, 1, 1, H_GNN), lambda nb, i: (nb, i, 0, 0)),
        out_shape=jax.ShapeDtypeStruct((NB, G, 1, H_GNN), jnp.float32),
    )


def _gru_heads_body(g_ref, s_ref, t_ref,
                    wihg, whhg, bihg, bhhg,
                    wihs, whhs, bihs, bhhs,
                    wiht, whht, biht, bhht,
                    wfc, bfc, wst, bst, wca, bca,
                    stim_ref, cause_ref):
    H = H_FC

    def gru(seq, wih, whh, bih, bhh):
        h = jnp.zeros((B, H), jnp.float32)
        hs = []
        for t in range(T):
            x = seq[:, t, :]
            gi = jnp.dot(x, wih[...], preferred_element_type=jnp.float32) + bih[...]
            gh = jnp.dot(h, whh[...], preferred_element_type=jnp.float32) + bhh[...]
            r = jax.nn.sigmoid(gi[:, 0:H] + gh[:, 0:H])
            z = jax.nn.sigmoid(gi[:, H:2 * H] + gh[:, H:2 * H])
            n = jnp.tanh(gi[:, 2 * H:3 * H] + r * gh[:, 2 * H:3 * H])
            h = (1.0 - z) * n + z * h
            hs.append(h)
        return hs

    hg = gru(jnp.sum(g_ref[...], axis=0), wihg, whhg, bihg, bhhg)
    hs_ = gru(s_ref[...], wihs, whhs, bihs, bhhs)
    ht = gru(t_ref[...], wiht, whht, biht, bhht)
    for t in range(T):
        cat = jnp.concatenate([hg[t], hs_[t], ht[t]], axis=1)
        hO = jnp.dot(cat, wfc[...], preferred_element_type=jnp.float32) + bfc[...]
        hO = jnp.maximum(hO, 0.0)
        stim_ref[:, t, :] = jnp.dot(hO, wst[...], preferred_element_type=jnp.float32) + bst[...]
        cause_ref[:, t, :] = jnp.dot(hO, wca[...], preferred_element_type=jnp.float32) + bca[...]


_gru_heads = pl.pallas_call(
    _gru_heads_body,
    out_shape=(jax.ShapeDtypeStruct((B, T, OUT_S), jnp.float32),
               jax.ShapeDtypeStruct((B, T, OUT_C), jnp.float32)),
)


def kernel(node_feats, edge_index, bSensor, bTarget, bArea,
           W_gnn1, b_gnn1, W_gnn3, b_gnn3,
           W_ih_G, W_hh_G, b_ih_G, b_hh_G,
           W_ih_S, W_hh_S, b_ih_S, b_hh_S,
           W_ih_T, W_hh_T, b_ih_T, b_hh_T,
           W_fc1, b_fc1, W_stim, b_stim, W_cause, b_cause):
    src_rows = edge_index[0].reshape(NW, CPT, CH)
    dst_rows = edge_index[1].reshape(NW, CPT, CH)

    x1 = node_feats.reshape(G, N, D_IN)
    y0a, y1a, deg0, deg1 = _make_sc_agg(True)(x1, src_rows, dst_rows)
    deg0 = deg0.reshape(N, H_GNN)
    deg1 = deg1.reshape(N, H_GNN)
    z1 = _tc_layer(True)(y0a.reshape(G, N, H_GNN), y1a.reshape(G, N, H_GNN),
                         deg0, deg1, W_gnn1.T, b_gnn1.reshape(1, -1))
    y0b, y1b = _make_sc_agg(False)(z1, src_rows, dst_rows)
    rp = _tc_layer(False)(y0b.reshape(G, N, H_GNN), y1b.reshape(G, N, H_GNN),
                          deg0, deg1, W_gnn3.T, b_gnn3.reshape(1, -1))

    stim4, cause4 = _gru_heads(
        rp.reshape(NB, B, T, H_RNN), bSensor, bTarget,
        W_ih_G.T, W_hh_G.T, b_ih_G.reshape(1, -1), b_hh_G.reshape(1, -1),
        W_ih_S.T, W_hh_S.T, b_ih_S.reshape(1, -1), b_hh_S.reshape(1, -1),
        W_ih_T.T, W_hh_T.T, b_ih_T.reshape(1, -1), b_hh_T.reshape(1, -1),
        W_fc1.T, b_fc1.reshape(1, -1),
        W_stim.T, b_stim.reshape(1, -1),
        W_cause.T, b_cause.reshape(1, -1))
    return (stim4.reshape(B * T, OUT_S), cause4.reshape(B * T, OUT_C))


# speedup vs baseline: 1.0875x; 1.0091x over previous
import functools

import jax
import jax.numpy as jnp
from jax import lax
from jax.experimental import pallas as pl
from jax.experimental.pallas import tpu as pltpu
from jax.experimental.pallas import tpu_sc as plsc

B, T, N, E = 4, 4, 10000, 160000
D_IN, H_GNN, H_RNN, H_FC = 128, 128, 128, 128
D_S, D_T, OUT_S, OUT_C = 64, 64, 10, 10
G = B * T
NC, NS = 2, 16
NW = NC * NS
EPT = E // NW
CH = 125
CPT = EPT // CH
U = 10
RPT = N // NS


def _sc_agg_body(with_deg, x_ref, src_ref, dst_ref, *rest):
    if with_deg:
        (y0_ref, y1_ref, d0_ref, d1_ref, src_v, dst_v, rows0, rows1,
         y_sh, gsem0, gsem1, ssem0, ssem1, w0, w1, w2, w3, w4) = rest
    else:
        (y0_ref, y1_ref, src_v, dst_v, rows0, rows1, y_sh,
         gsem0, gsem1, ssem0, ssem1, w0, w1, w2, w3, w4) = rest
    wsems = (w0, w1, w2, w3, w4)
    c = lax.axis_index("c")
    s = lax.axis_index("s")
    w = c * NS + s

    def _fill(buf, val):
        def _fb(t, carry):
            r = t // 8
            j = t % 8
            buf[r, pl.ds(j * 16, 16)] = jnp.full((16,), val, jnp.float32)
            return carry
        lax.fori_loop(0, CH * 8, _fb, 0)

    def _zero_own_rows():
        _fill(rows0, 0.0)
        for j in range(RPT // CH):
            pltpu.sync_copy(rows0, y_sh.at[pl.ds(s * RPT + j * CH, CH)])

    def _scatter(buf, k):
        pltpu.sync_copy(buf, y_sh.at[dst_v.at[k]], add=True)

    pltpu.sync_copy(dst_ref.at[w], dst_v)
    pltpu.sync_copy(src_ref.at[w], src_v)

    if with_deg:
        _zero_own_rows()
        _fill(rows0, 1.0)
        plsc.subcore_barrier()

        def _dchunk(k, carry):
            pltpu.sync_copy(rows0, y_sh.at[dst_v.at[k]], add=True)
            return carry
        lax.fori_loop(0, CPT, _dchunk, 0)
        plsc.subcore_barrier()

        @pl.when(c == 0)
        def _():
            pltpu.sync_copy(y_sh.at[pl.ds(s * RPT, RPT)], d0_ref.at[s])

        @pl.when(c == 1)
        def _():
            pltpu.sync_copy(y_sh.at[pl.ds(s * RPT, RPT)], d1_ref.at[s])
        _zero_own_rows()

    if not with_deg:
        _zero_own_rows()

    def _inst(i, carry):
        plsc.subcore_barrier()

        bufs = (rows0, rows1)
        gsems = (gsem0, gsem1)
        ssems = (ssem0, ssem1)

        def _block(p, carry2):
            base = p * U
            gd = [None] * U
            sd = [None] * U
            gd[0] = pltpu.async_copy(
                x_ref.at[i].at[src_v.at[base]], bufs[0], gsems[0])
            for j in range(U):
                gd[j].wait()
                sd[j] = pltpu.async_copy(
                    bufs[j % 2], y_sh.at[dst_v.at[base + j]],
                    ssems[j % 2], add=True)
                if j + 1 < U:
                    if j >= 1:
                        sd[j - 1].wait()
                    gd[j + 1] = pltpu.async_copy(
                        x_ref.at[i].at[src_v.at[base + j + 1]],
                        bufs[(j + 1) % 2], gsems[(j + 1) % 2])
            sd[U - 2].wait()
            sd[U - 1].wait()
            return carry2
        lax.fori_loop(0, CPT // U, _block, 0)
        plsc.subcore_barrier()
        _fill(rows0, 0.0)

        def _tail(yref):
            wds = [pltpu.async_copy(y_sh.at[pl.ds(s * RPT + j * CH, CH)],
                                    yref.at[i, s, j], wsems[j])
                   for j in range(RPT // CH)]
            for j in range(RPT // CH):
                wds[j].wait()
                pltpu.sync_copy(rows0, y_sh.at[pl.ds(s * RPT + j * CH, CH)])

        @pl.when(c == 0)
        def _():
            _tail(y0_ref)

        @pl.when(c == 1)
        def _():
            _tail(y1_ref)
        return carry
    lax.fori_loop(0, G, _inst, 0)


def _make_sc_agg(with_deg):
    yshape = jax.ShapeDtypeStruct((G, NS, RPT // CH, CH, H_GNN), jnp.float32)
    dshape = jax.ShapeDtypeStruct((NS, RPT, H_GNN), jnp.float32)
    out_type = (yshape, yshape, dshape, dshape) if with_deg else (yshape, yshape)
    scratch = [
        pltpu.VMEM((CPT, CH), jnp.int32),
        pltpu.VMEM((CPT, CH), jnp.int32),
        pltpu.VMEM((CH, H_GNN), jnp.float32),
        pltpu.VMEM((CH, H_GNN), jnp.float32),
    ]
    scratch += [pltpu.VMEM_SHARED((N, H_GNN), jnp.float32)]
    scratch += [pltpu.SemaphoreType.DMA] * 9
    mesh = plsc.VectorSubcoreMesh(core_axis_name="c", subcore_axis_name="s")
    return pl.kernel(
        functools.partial(_sc_agg_body, with_deg),
        out_type=out_type,
        mesh=mesh,
        scratch_types=scratch,
    )


BN = 2000
NB = N // BN


def _tc_layer1_body(y0, y1, d0, d1, wt, b, z_out):
    y = y0[0] + y1[0]
    deg = d0[:, 0:1] + d1[:, 0:1]
    m = y * (1.0 / jnp.maximum(deg, 1.0))
    z = jnp.dot(m, wt[...], preferred_element_type=jnp.float32) + b[...]
    z_out[0] = jnp.maximum(z, 0.0)


def _tc_layer2_body(y0, y1, d0, d1, wt, b, r_out):
    y = y0[0] + y1[0]
    deg = d0[:, 0:1] + d1[:, 0:1]
    m = y * (1.0 / jnp.maximum(deg, 1.0))
    z = jnp.dot(m, wt[...], preferred_element_type=jnp.float32) + b[...]
    z = jnp.maximum(z, 0.0)
    r_out[...] = (jnp.sum(z, axis=0, keepdims=True) * (1.0 / N))[None, None]


def _tc_layer(emit_z):
    in_specs = [
        pl.BlockSpec((1, BN, H_GNN), lambda nb, i: (i, nb, 0)),
        pl.BlockSpec((1, BN, H_GNN), lambda nb, i: (i, nb, 0)),
        pl.BlockSpec((BN, H_GNN), lambda nb, i: (nb, 0)),
        pl.BlockSpec((BN, H_GNN), lambda nb, i: (nb, 0)),
        pl.BlockSpec((H_GNN, H_GNN), lambda nb, i: (0, 0)),
        pl.BlockSpec((1, H_GNN), lambda nb, i: (0, 0)),
    ]
    if emit_z:
        return pl.pallas_call(
            _tc_layer1_body,
            grid=(NB, G),
            in_specs=in_specs,
            out_specs=pl.BlockSpec((1, BN, H_GNN), lambda nb, i: (i, nb, 0)),
            out_shape=jax.ShapeDtypeStruct((G, N, H_GNN), jnp.float32),
        )
    return pl.pallas_call(
        _tc_layer2_body,
        grid=(NB, G),
        in_specs=in_specs,
        out_specs=pl.BlockSpec((1, 1, 1, H_GNN), lambda nb, i: (nb, i, 0, 0)),
        out_shape=jax.ShapeDtypeStruct((NB, G, 1, H_GNN), jnp.float32),
    )


def _gru_heads_body(g_ref, s_ref, t_ref,
                    wihg, whhg, bihg, bhhg,
                    wihs, whhs, bihs, bhhs,
                    wiht, whht, biht, bhht,
                    wfc, bfc, wst, bst, wca, bca,
                    stim_ref, cause_ref):
    H = H_FC

    def gru(seq, wih, whh, bih, bhh):
        h = jnp.zeros((B, H), jnp.float32)
        hs = []
        for t in range(T):
            x = seq[:, t, :]
            gi = jnp.dot(x, wih[...], preferred_element_type=jnp.float32) + bih[...]
            gh = jnp.dot(h, whh[...], preferred_element_type=jnp.float32) + bhh[...]
            r = jax.nn.sigmoid(gi[:, 0:H] + gh[:, 0:H])
            z = jax.nn.sigmoid(gi[:, H:2 * H] + gh[:, H:2 * H])
            n = jnp.tanh(gi[:, 2 * H:3 * H] + r * gh[:, 2 * H:3 * H])
            h = (1.0 - z) * n + z * h
            hs.append(h)
        return hs

    hg = gru(jnp.sum(g_ref[...], axis=0), wihg, whhg, bihg, bhhg)
    hs_ = gru(s_ref[...], wihs, whhs, bihs, bhhs)
    ht = gru(t_ref[...], wiht, whht, biht, bhht)
    for t in range(T):
        cat = jnp.concatenate([hg[t], hs_[t], ht[t]], axis=1)
        hO = jnp.dot(cat, wfc[...], preferred_element_type=jnp.float32) + bfc[...]
        hO = jnp.maximum(hO, 0.0)
        stim_ref[:, t, :] = jnp.dot(hO, wst[...], preferred_element_type=jnp.float32) + bst[...]
        cause_ref[:, t, :] = jnp.dot(hO, wca[...], preferred_element_type=jnp.float32) + bca[...]


_gru_heads = pl.pallas_call(
    _gru_heads_body,
    out_shape=(jax.ShapeDtypeStruct((B, T, OUT_S), jnp.float32),
               jax.ShapeDtypeStruct((B, T, OUT_C), jnp.float32)),
)


def kernel(node_feats, edge_index, bSensor, bTarget, bArea,
           W_gnn1, b_gnn1, W_gnn3, b_gnn3,
           W_ih_G, W_hh_G, b_ih_G, b_hh_G,
           W_ih_S, W_hh_S, b_ih_S, b_hh_S,
           W_ih_T, W_hh_T, b_ih_T, b_hh_T,
           W_fc1, b_fc1, W_stim, b_stim, W_cause, b_cause):
    src_rows = edge_index[0].reshape(NW, CPT, CH)
    dst_rows = edge_index[1].reshape(NW, CPT, CH)

    x1 = node_feats.reshape(G, N, D_IN)
    y0a, y1a, deg0, deg1 = _make_sc_agg(True)(x1, src_rows, dst_rows)
    deg0 = deg0.reshape(N, H_GNN)
    deg1 = deg1.reshape(N, H_GNN)
    z1 = _tc_layer(True)(y0a.reshape(G, N, H_GNN), y1a.reshape(G, N, H_GNN),
                         deg0, deg1, W_gnn1.T, b_gnn1.reshape(1, -1))
    y0b, y1b = _make_sc_agg(False)(z1, src_rows, dst_rows)
    rp = _tc_layer(False)(y0b.reshape(G, N, H_GNN), y1b.reshape(G, N, H_GNN),
                          deg0, deg1, W_gnn3.T, b_gnn3.reshape(1, -1))

    stim4, cause4 = _gru_heads(
        rp.reshape(NB, B, T, H_RNN), bSensor, bTarget,
        W_ih_G.T, W_hh_G.T, b_ih_G.reshape(1, -1), b_hh_G.reshape(1, -1),
        W_ih_S.T, W_hh_S.T, b_ih_S.reshape(1, -1), b_hh_S.reshape(1, -1),
        W_ih_T.T, W_hh_T.T, b_ih_T.reshape(1, -1), b_hh_T.reshape(1, -1),
        W_fc1.T, b_fc1.reshape(1, -1),
        W_stim.T, b_stim.reshape(1, -1),
        W_cause.T, b_cause.reshape(1, -1))
    return (stim4.reshape(B * T, OUT_S), cause4.reshape(B * T, OUT_C))


# speedup vs baseline: 1.0944x; 1.0063x over previous
import functools

import jax
import jax.numpy as jnp
from jax import lax
from jax.experimental import pallas as pl
from jax.experimental.pallas import tpu as pltpu
from jax.experimental.pallas import tpu_sc as plsc

B, T, N, E = 4, 4, 10000, 160000
D_IN, H_GNN, H_RNN, H_FC = 128, 128, 128, 128
D_S, D_T, OUT_S, OUT_C = 64, 64, 10, 10
G = B * T
NC, NS = 2, 16
NW = NC * NS
EPT = E // NW
CH = 125
CPT = EPT // CH
U = 10
RPT = N // NS


def _sc_agg_body(with_deg, x_ref, src_ref, dst_ref, *rest):
    if with_deg:
        (y0_ref, y1_ref, d0_ref, d1_ref, src_v, dst_v, rows0, rows1,
         y_sh, gsem0, gsem1, ssem0, ssem1, w0, w1, w2, w3, w4) = rest
    else:
        (y0_ref, y1_ref, src_v, dst_v, rows0, rows1, y_sh,
         gsem0, gsem1, ssem0, ssem1, w0, w1, w2, w3, w4) = rest
    wsems = (w0, w1, w2, w3, w4)
    c = lax.axis_index("c")
    s = lax.axis_index("s")
    w = c * NS + s

    def _fill(buf, val):
        def _fb(t, carry):
            r = t // 8
            j = t % 8
            buf[r, pl.ds(j * 16, 16)] = jnp.full((16,), val, jnp.float32)
            return carry
        lax.fori_loop(0, CH * 8, _fb, 0)

    def _zero_own_rows():
        _fill(rows0, 0.0)
        for j in range(RPT // CH):
            pltpu.sync_copy(rows0, y_sh.at[pl.ds(s * RPT + j * CH, CH)])

    def _scatter(buf, k):
        pltpu.sync_copy(buf, y_sh.at[dst_v.at[k]], add=True)

    pltpu.sync_copy(dst_ref.at[w], dst_v)
    pltpu.sync_copy(src_ref.at[w], src_v)

    if with_deg:
        _zero_own_rows()
        _fill(rows0, 1.0)
        plsc.subcore_barrier()

        def _dchunk(k, carry):
            pltpu.sync_copy(rows0, y_sh.at[dst_v.at[k]], add=True)
            return carry
        lax.fori_loop(0, CPT, _dchunk, 0)
        plsc.subcore_barrier()

        @pl.when(c == 0)
        def _():
            pltpu.sync_copy(y_sh.at[pl.ds(s * RPT, RPT)], d0_ref.at[s])

        @pl.when(c == 1)
        def _():
            pltpu.sync_copy(y_sh.at[pl.ds(s * RPT, RPT)], d1_ref.at[s])

    def _inst(i, carry):
        _zero_own_rows()
        plsc.subcore_barrier()

        bufs = (rows0, rows1)
        gsems = (gsem0, gsem1)
        ssems = (ssem0, ssem1)

        def _block(p, carry2):
            base = p * U
            gd = [None] * U
            sd = [None] * U
            gd[0] = pltpu.async_copy(
                x_ref.at[i].at[src_v.at[base]], bufs[0], gsems[0])
            for j in range(U):
                gd[j].wait()
                sd[j] = pltpu.async_copy(
                    bufs[j % 2], y_sh.at[dst_v.at[base + j]],
                    ssems[j % 2], add=True)
                if j + 1 < U:
                    if j >= 1:
                        sd[j - 1].wait()
                    gd[j + 1] = pltpu.async_copy(
                        x_ref.at[i].at[src_v.at[base + j + 1]],
                        bufs[(j + 1) % 2], gsems[(j + 1) % 2])
            sd[U - 2].wait()
            sd[U - 1].wait()
            return carry2
        lax.fori_loop(0, CPT // U, _block, 0)
        plsc.subcore_barrier()

        @pl.when(c == 0)
        def _():
            for j in range(RPT // CH):
                pltpu.sync_copy(y_sh.at[pl.ds(s * RPT + j * CH, CH)],
                                y0_ref.at[i, s, j])

        @pl.when(c == 1)
        def _():
            for j in range(RPT // CH):
                pltpu.sync_copy(y_sh.at[pl.ds(s * RPT + j * CH, CH)],
                                y1_ref.at[i, s, j])
        plsc.subcore_barrier()
        return carry
    lax.fori_loop(0, G, _inst, 0)


def _make_sc_agg(with_deg):
    yshape = jax.ShapeDtypeStruct((G, NS, RPT // CH, CH, H_GNN), jnp.float32)
    dshape = jax.ShapeDtypeStruct((NS, RPT, H_GNN), jnp.float32)
    out_type = (yshape, yshape, dshape, dshape) if with_deg else (yshape, yshape)
    scratch = [
        pltpu.VMEM((CPT, CH), jnp.int32),
        pltpu.VMEM((CPT, CH), jnp.int32),
        pltpu.VMEM((CH, H_GNN), jnp.float32),
        pltpu.VMEM((CH, H_GNN), jnp.float32),
    ]
    scratch += [pltpu.VMEM_SHARED((N, H_GNN), jnp.float32)]
    scratch += [pltpu.SemaphoreType.DMA] * 9
    mesh = plsc.VectorSubcoreMesh(core_axis_name="c", subcore_axis_name="s")
    return pl.kernel(
        functools.partial(_sc_agg_body, with_deg),
        out_type=out_type,
        mesh=mesh,
        scratch_types=scratch,
    )


BN = 2000
NB = N // BN


def _tc_layer1_body(y0, y1, d0, d1, wt, b, z_out):
    y = y0[0] + y1[0]
    deg = d0[:, 0:1] + d1[:, 0:1]
    m = y * (1.0 / jnp.maximum(deg, 1.0))
    z = jnp.dot(m, wt[...], preferred_element_type=jnp.float32) + b[...]
    z_out[0] = jnp.maximum(z, 0.0)


def _tc_layer2_body(y0, y1, d0, d1, wt, b, r_out):
    y = y0[0] + y1[0]
    deg = d0[:, 0:1] + d1[:, 0:1]
    m = y * (1.0 / jnp.maximum(deg, 1.0))
    z = jnp.dot(m, wt[...], preferred_element_type=jnp.float32) + b[...]
    z = jnp.maximum(z, 0.0)
    r_out[...] = (jnp.sum(z, axis=0, keepdims=True) * (1.0 / N))[None, None]


def _tc_layer(emit_z):
    in_specs = [
        pl.BlockSpec((1, BN, H_GNN), lambda nb, i: (i, nb, 0)),
        pl.BlockSpec((1, BN, H_GNN), lambda nb, i: (i, nb, 0)),
        pl.BlockSpec((BN, H_GNN), lambda nb, i: (nb, 0)),
        pl.BlockSpec((BN, H_GNN), lambda nb, i: (nb, 0)),
        pl.BlockSpec((H_GNN, H_GNN), lambda nb, i: (0, 0)),
        pl.BlockSpec((1, H_GNN), lambda nb, i: (0, 0)),
    ]
    if emit_z:
        return pl.pallas_call(
            _tc_layer1_body,
            grid=(NB, G),
            in_specs=in_specs,
            out_specs=pl.BlockSpec((1, BN, H_GNN), lambda nb, i: (i, nb, 0)),
            out_shape=jax.ShapeDtypeStruct((G, N, H_GNN), jnp.float32),
        )
    return pl.pallas_call(
        _tc_layer2_body,
        grid=(NB, G),
        in_specs=in_specs,
        out_specs=pl.BlockSpec((1, 1, 1, H_GNN), lambda nb, i: (nb, i, 0, 0)),
        out_shape=jax.ShapeDtypeStruct((NB, G, 1, H_GNN), jnp.float32),
    )


def _gru_heads_body(g_ref, s_ref, t_ref,
                    wihg, whhg, bihg, bhhg,
                    wihs, whhs, bihs, bhhs,
                    wiht, whht, biht, bhht,
                    wfc, bfc, wst, bst, wca, bca,
                    stim_ref, cause_ref):
    H = H_FC

    def gru(seq, wih, whh, bih, bhh):
        h = jnp.zeros((B, H), jnp.float32)
        hs = []
        for t in range(T):
            x = seq[:, t, :]
            gi = jnp.dot(x, wih[...], preferred_element_type=jnp.float32) + bih[...]
            gh = jnp.dot(h, whh[...], preferred_element_type=jnp.float32) + bhh[...]
            r = jax.nn.sigmoid(gi[:, 0:H] + gh[:, 0:H])
            z = jax.nn.sigmoid(gi[:, H:2 * H] + gh[:, H:2 * H])
            n = jnp.tanh(gi[:, 2 * H:3 * H] + r * gh[:, 2 * H:3 * H])
            h = (1.0 - z) * n + z * h
            hs.append(h)
        return hs

    hg = gru(jnp.sum(g_ref[...], axis=0), wihg, whhg, bihg, bhhg)
    hs_ = gru(s_ref[...], wihs, whhs, bihs, bhhs)
    ht = gru(t_ref[...], wiht, whht, biht, bhht)
    for t in range(T):
        cat = jnp.concatenate([hg[t], hs_[t], ht[t]], axis=1)
        hO = jnp.dot(cat, wfc[...], preferred_element_type=jnp.float32) + bfc[...]
        hO = jnp.maximum(hO, 0.0)
        stim_ref[:, t, :] = jnp.dot(hO, wst[...], preferred_element_type=jnp.float32) + bst[...]
        cause_ref[:, t, :] = jnp.dot(hO, wca[...], preferred_element_type=jnp.float32) + bca[...]


_gru_heads = pl.pallas_call(
    _gru_heads_body,
    out_shape=(jax.ShapeDtypeStruct((B, T, OUT_S), jnp.float32),
               jax.ShapeDtypeStruct((B, T, OUT_C), jnp.float32)),
)


def kernel(node_feats, edge_index, bSensor, bTarget, bArea,
           W_gnn1, b_gnn1, W_gnn3, b_gnn3,
           W_ih_G, W_hh_G, b_ih_G, b_hh_G,
           W_ih_S, W_hh_S, b_ih_S, b_hh_S,
           W_ih_T, W_hh_T, b_ih_T, b_hh_T,
           W_fc1, b_fc1, W_stim, b_stim, W_cause, b_cause):
    src_rows = edge_index[0].reshape(NW, CPT, CH)
    dst_rows = edge_index[1].reshape(NW, CPT, CH)

    x1 = node_feats.reshape(G, N, D_IN)
    y0a, y1a, deg0, deg1 = _make_sc_agg(True)(x1, src_rows, dst_rows)
    deg0 = deg0.reshape(N, H_GNN)
    deg1 = deg1.reshape(N, H_GNN)
    z1 = _tc_layer(True)(y0a.reshape(G, N, H_GNN), y1a.reshape(G, N, H_GNN),
                         deg0, deg1, W_gnn1.T, b_gnn1.reshape(1, -1))
    y0b, y1b = _make_sc_agg(False)(z1, src_rows, dst_rows)
    rp = _tc_layer(False)(y0b.reshape(G, N, H_GNN), y1b.reshape(G, N, H_GNN),
                          deg0, deg1, W_gnn3.T, b_gnn3.reshape(1, -1))

    stim4, cause4 = _gru_heads(
        rp.reshape(NB, B, T, H_RNN), bSensor, bTarget,
        W_ih_G.T, W_hh_G.T, b_ih_G.reshape(1, -1), b_hh_G.reshape(1, -1),
        W_ih_S.T, W_hh_S.T, b_ih_S.reshape(1, -1), b_hh_S.reshape(1, -1),
        W_ih_T.T, W_hh_T.T, b_ih_T.reshape(1, -1), b_hh_T.reshape(1, -1),
        W_fc1.T, b_fc1.reshape(1, -1),
        W_stim.T, b_stim.reshape(1, -1),
        W_cause.T, b_cause.reshape(1, -1))
    return (stim4.reshape(B * T, OUT_S), cause4.reshape(B * T, OUT_C))


# speedup vs baseline: 1.1929x; 1.0900x over previous
import functools

import jax
import jax.numpy as jnp
from jax import lax
from jax.experimental import pallas as pl
from jax.experimental.pallas import tpu as pltpu
from jax.experimental.pallas import tpu_sc as plsc

B, T, N, E = 4, 4, 10000, 160000
D_IN, H_GNN, H_RNN, H_FC = 128, 128, 128, 128
D_S, D_T, OUT_S, OUT_C = 64, 64, 10, 10
G = B * T
NC, NS = 2, 16
NW = NC * NS
EPT = E // NW
CH = 125
CPT = EPT // CH
U = 10
RPT = N // NS


def _sc_agg_body(with_deg, g, x_ref, src_ref, dst_ref, *rest):
    if with_deg:
        (y0_ref, y1_ref, d0_ref, d1_ref, src_v, dst_v, rows0, rows1,
         y_sh, gsem0, gsem1, ssem0, ssem1, w0, w1, w2, w3, w4) = rest
    else:
        (y0_ref, y1_ref, src_v, dst_v, rows0, rows1, y_sh,
         gsem0, gsem1, ssem0, ssem1, w0, w1, w2, w3, w4) = rest
    wsems = (w0, w1, w2, w3, w4)
    c = lax.axis_index("c")
    s = lax.axis_index("s")
    w = c * NS + s

    def _fill(buf, val):
        def _fb(t, carry):
            r = t // 8
            j = t % 8
            buf[r, pl.ds(j * 16, 16)] = jnp.full((16,), val, jnp.float32)
            return carry
        lax.fori_loop(0, CH * 8, _fb, 0)

    def _zero_own_rows():
        _fill(rows0, 0.0)
        for j in range(RPT // CH):
            pltpu.sync_copy(rows0, y_sh.at[pl.ds(s * RPT + j * CH, CH)])

    def _scatter(buf, k):
        pltpu.sync_copy(buf, y_sh.at[dst_v.at[k]], add=True)

    pltpu.sync_copy(dst_ref.at[w], dst_v)
    pltpu.sync_copy(src_ref.at[w], src_v)

    if with_deg:
        _zero_own_rows()
        _fill(rows0, 1.0)
        plsc.subcore_barrier()

        def _dchunk(k, carry):
            pltpu.sync_copy(rows0, y_sh.at[dst_v.at[k]], add=True)
            return carry
        lax.fori_loop(0, CPT, _dchunk, 0)
        plsc.subcore_barrier()

        @pl.when(c == 0)
        def _():
            pltpu.sync_copy(y_sh.at[pl.ds(s * RPT, RPT)], d0_ref.at[s])

        @pl.when(c == 1)
        def _():
            pltpu.sync_copy(y_sh.at[pl.ds(s * RPT, RPT)], d1_ref.at[s])

    def _inst(i, carry):
        _zero_own_rows()
        plsc.subcore_barrier()

        bufs = (rows0, rows1)
        gsems = (gsem0, gsem1)
        ssems = (ssem0, ssem1)

        def _block(p, carry2):
            base = p * U
            gd = [None] * U
            sd = [None] * U
            gd[0] = pltpu.async_copy(
                x_ref.at[i].at[src_v.at[base]], bufs[0], gsems[0])
            for j in range(U):
                gd[j].wait()
                sd[j] = pltpu.async_copy(
                    bufs[j % 2], y_sh.at[dst_v.at[base + j]],
                    ssems[j % 2], add=True)
                if j + 1 < U:
                    if j >= 1:
                        sd[j - 1].wait()
                    gd[j + 1] = pltpu.async_copy(
                        x_ref.at[i].at[src_v.at[base + j + 1]],
                        bufs[(j + 1) % 2], gsems[(j + 1) % 2])
            sd[U - 2].wait()
            sd[U - 1].wait()
            return carry2
        lax.fori_loop(0, CPT // U, _block, 0)
        plsc.subcore_barrier()

        @pl.when(c == 0)
        def _():
            for j in range(RPT // CH):
                pltpu.sync_copy(y_sh.at[pl.ds(s * RPT + j * CH, CH)],
                                y0_ref.at[i, s, j])

        @pl.when(c == 1)
        def _():
            for j in range(RPT // CH):
                pltpu.sync_copy(y_sh.at[pl.ds(s * RPT + j * CH, CH)],
                                y1_ref.at[i, s, j])
        plsc.subcore_barrier()
        return carry
    lax.fori_loop(0, g, _inst, 0)


def _make_sc_agg(with_deg, g):
    yshape = jax.ShapeDtypeStruct((g, NS, RPT // CH, CH, H_GNN), jnp.float32)
    dshape = jax.ShapeDtypeStruct((NS, RPT, H_GNN), jnp.float32)
    out_type = (yshape, yshape, dshape, dshape) if with_deg else (yshape, yshape)
    scratch = [
        pltpu.VMEM((CPT, CH), jnp.int32),
        pltpu.VMEM((CPT, CH), jnp.int32),
        pltpu.VMEM((CH, H_GNN), jnp.float32),
        pltpu.VMEM((CH, H_GNN), jnp.float32),
    ]
    scratch += [pltpu.VMEM_SHARED((N, H_GNN), jnp.float32)]
    scratch += [pltpu.SemaphoreType.DMA] * 9
    mesh = plsc.VectorSubcoreMesh(core_axis_name="c", subcore_axis_name="s")
    return pl.kernel(
        functools.partial(_sc_agg_body, with_deg, g),
        out_type=out_type,
        mesh=mesh,
        scratch_types=scratch,
    )


BN = 2000
NB = N // BN


def _tc_layer1_body(y0, y1, d0, d1, wt, b, z_out):
    y = y0[0] + y1[0]
    deg = d0[:, 0:1] + d1[:, 0:1]
    m = y * (1.0 / jnp.maximum(deg, 1.0))
    z = jnp.dot(m, wt[...], preferred_element_type=jnp.float32) + b[...]
    z_out[0] = jnp.maximum(z, 0.0)


def _tc_layer2_body(y0, y1, d0, d1, wt, b, r_out):
    y = y0[0] + y1[0]
    deg = d0[:, 0:1] + d1[:, 0:1]
    m = y * (1.0 / jnp.maximum(deg, 1.0))
    z = jnp.dot(m, wt[...], preferred_element_type=jnp.float32) + b[...]
    z = jnp.maximum(z, 0.0)
    r_out[...] = (jnp.sum(z, axis=0, keepdims=True) * (1.0 / N))[None, None]


def _tc_layer(emit_z, g):
    in_specs = [
        pl.BlockSpec((1, BN, H_GNN), lambda nb, i: (i, nb, 0)),
        pl.BlockSpec((1, BN, H_GNN), lambda nb, i: (i, nb, 0)),
        pl.BlockSpec((BN, H_GNN), lambda nb, i: (nb, 0)),
        pl.BlockSpec((BN, H_GNN), lambda nb, i: (nb, 0)),
        pl.BlockSpec((H_GNN, H_GNN), lambda nb, i: (0, 0)),
        pl.BlockSpec((1, H_GNN), lambda nb, i: (0, 0)),
    ]
    if emit_z:
        return pl.pallas_call(
            _tc_layer1_body,
            grid=(NB, g),
            in_specs=in_specs,
            out_specs=pl.BlockSpec((1, BN, H_GNN), lambda nb, i: (i, nb, 0)),
            out_shape=jax.ShapeDtypeStruct((g, N, H_GNN), jnp.float32),
        )
    return pl.pallas_call(
        _tc_layer2_body,
        grid=(NB, g),
        in_specs=in_specs,
        out_specs=pl.BlockSpec((1, 1, 1, H_GNN), lambda nb, i: (nb, i, 0, 0)),
        out_shape=jax.ShapeDtypeStruct((NB, g, 1, H_GNN), jnp.float32),
    )


def _gru_heads_body(g_ref, s_ref, t_ref,
                    wihg, whhg, bihg, bhhg,
                    wihs, whhs, bihs, bhhs,
                    wiht, whht, biht, bhht,
                    wfc, bfc, wst, bst, wca, bca,
                    stim_ref, cause_ref):
    H = H_FC

    def gru(seq, wih, whh, bih, bhh):
        h = jnp.zeros((B, H), jnp.float32)
        hs = []
        for t in range(T):
            x = seq[:, t, :]
            gi = jnp.dot(x, wih[...], preferred_element_type=jnp.float32) + bih[...]
            gh = jnp.dot(h, whh[...], preferred_element_type=jnp.float32) + bhh[...]
            r = jax.nn.sigmoid(gi[:, 0:H] + gh[:, 0:H])
            z = jax.nn.sigmoid(gi[:, H:2 * H] + gh[:, H:2 * H])
            n = jnp.tanh(gi[:, 2 * H:3 * H] + r * gh[:, 2 * H:3 * H])
            h = (1.0 - z) * n + z * h
            hs.append(h)
        return hs

    hg = gru(jnp.sum(g_ref[...], axis=0), wihg, whhg, bihg, bhhg)
    hs_ = gru(s_ref[...], wihs, whhs, bihs, bhhs)
    ht = gru(t_ref[...], wiht, whht, biht, bhht)
    for t in range(T):
        cat = jnp.concatenate([hg[t], hs_[t], ht[t]], axis=1)
        hO = jnp.dot(cat, wfc[...], preferred_element_type=jnp.float32) + bfc[...]
        hO = jnp.maximum(hO, 0.0)
        stim_ref[:, t, :] = jnp.dot(hO, wst[...], preferred_element_type=jnp.float32) + bst[...]
        cause_ref[:, t, :] = jnp.dot(hO, wca[...], preferred_element_type=jnp.float32) + bca[...]


_gru_heads = pl.pallas_call(
    _gru_heads_body,
    out_shape=(jax.ShapeDtypeStruct((B, T, OUT_S), jnp.float32),
               jax.ShapeDtypeStruct((B, T, OUT_C), jnp.float32)),
)


def kernel(node_feats, edge_index, bSensor, bTarget, bArea,
           W_gnn1, b_gnn1, W_gnn3, b_gnn3,
           W_ih_G, W_hh_G, b_ih_G, b_hh_G,
           W_ih_S, W_hh_S, b_ih_S, b_hh_S,
           W_ih_T, W_hh_T, b_ih_T, b_hh_T,
           W_fc1, b_fc1, W_stim, b_stim, W_cause, b_cause):
    src_rows = edge_index[0].reshape(NW, CPT, CH)
    dst_rows = edge_index[1].reshape(NW, CPT, CH)

    GH = G // 2
    x1 = node_feats.reshape(G, N, D_IN)
    xh = (x1[:GH], x1[GH:])
    wt1, bb1 = W_gnn1.T, b_gnn1.reshape(1, -1)
    wt3, bb3 = W_gnn3.T, b_gnn3.reshape(1, -1)

    y0a0, y1a0, deg0, deg1 = _make_sc_agg(True, GH)(xh[0], src_rows, dst_rows)
    deg0 = deg0.reshape(N, H_GNN)
    deg1 = deg1.reshape(N, H_GNN)
    y0a1, y1a1 = _make_sc_agg(False, GH)(xh[1], src_rows, dst_rows)
    z10 = _tc_layer(True, GH)(y0a0.reshape(GH, N, H_GNN),
                              y1a0.reshape(GH, N, H_GNN), deg0, deg1, wt1, bb1)
    y0b0, y1b0 = _make_sc_agg(False, GH)(z10, src_rows, dst_rows)
    z11 = _tc_layer(True, GH)(y0a1.reshape(GH, N, H_GNN),
                              y1a1.reshape(GH, N, H_GNN), deg0, deg1, wt1, bb1)
    rp0 = _tc_layer(False, GH)(y0b0.reshape(GH, N, H_GNN),
                               y1b0.reshape(GH, N, H_GNN), deg0, deg1, wt3, bb3)
    y0b1, y1b1 = _make_sc_agg(False, GH)(z11, src_rows, dst_rows)
    rp1 = _tc_layer(False, GH)(y0b1.reshape(GH, N, H_GNN),
                               y1b1.reshape(GH, N, H_GNN), deg0, deg1, wt3, bb3)
    rp = jnp.concatenate([rp0, rp1], axis=1)

    stim4, cause4 = _gru_heads(
        rp.reshape(NB, B, T, H_RNN), bSensor, bTarget,
        W_ih_G.T, W_hh_G.T, b_ih_G.reshape(1, -1), b_hh_G.reshape(1, -1),
        W_ih_S.T, W_hh_S.T, b_ih_S.reshape(1, -1), b_hh_S.reshape(1, -1),
        W_ih_T.T, W_hh_T.T, b_ih_T.reshape(1, -1), b_hh_T.reshape(1, -1),
        W_fc1.T, b_fc1.reshape(1, -1),
        W_stim.T, b_stim.reshape(1, -1),
        W_cause.T, b_cause.reshape(1, -1))
    return (stim4.reshape(B * T, OUT_S), cause4.reshape(B * T, OUT_C))


# speedup vs baseline: 1.2708x; 1.0654x over previous
import functools

import jax
import jax.numpy as jnp
from jax import lax
from jax.experimental import pallas as pl
from jax.experimental.pallas import tpu as pltpu
from jax.experimental.pallas import tpu_sc as plsc

B, T, N, E = 4, 4, 10000, 160000
D_IN, H_GNN, H_RNN, H_FC = 128, 128, 128, 128
D_S, D_T, OUT_S, OUT_C = 64, 64, 10, 10
G = B * T
NC, NS = 2, 16
NW = NC * NS
EPT = E // NW
CH = 125
CPT = EPT // CH
U = 10
RPT = N // NS


def _sc_agg_body(with_deg, g, x_ref, src_ref, dst_ref, *rest):
    if with_deg:
        (y0_ref, y1_ref, d0_ref, d1_ref, src_v, dst_v, rows0, rows1,
         y_sh, gsem0, gsem1, ssem0, ssem1, w0, w1, w2, w3, w4) = rest
    else:
        (y0_ref, y1_ref, src_v, dst_v, rows0, rows1, y_sh,
         gsem0, gsem1, ssem0, ssem1, w0, w1, w2, w3, w4) = rest
    wsems = (w0, w1, w2, w3, w4)
    c = lax.axis_index("c")
    s = lax.axis_index("s")
    w = c * NS + s

    def _fill(buf, val):
        def _fb(t, carry):
            r = t // 8
            j = t % 8
            buf[r, pl.ds(j * 16, 16)] = jnp.full((16,), val, jnp.float32)
            return carry
        lax.fori_loop(0, CH * 8, _fb, 0)

    def _zero_own_rows():
        _fill(rows0, 0.0)
        for j in range(RPT // CH):
            pltpu.sync_copy(rows0, y_sh.at[pl.ds(s * RPT + j * CH, CH)])

    def _scatter(buf, k):
        pltpu.sync_copy(buf, y_sh.at[dst_v.at[k]], add=True)

    pltpu.sync_copy(dst_ref.at[w], dst_v)
    pltpu.sync_copy(src_ref.at[w], src_v)

    if with_deg:
        _zero_own_rows()
        _fill(rows0, 1.0)
        plsc.subcore_barrier()

        def _dchunk(k, carry):
            pltpu.sync_copy(rows0, y_sh.at[dst_v.at[k]], add=True)
            return carry
        lax.fori_loop(0, CPT, _dchunk, 0)
        plsc.subcore_barrier()

        @pl.when(c == 0)
        def _():
            pltpu.sync_copy(y_sh.at[pl.ds(s * RPT, RPT)], d0_ref.at[s])

        @pl.when(c == 1)
        def _():
            pltpu.sync_copy(y_sh.at[pl.ds(s * RPT, RPT)], d1_ref.at[s])

    def _inst(i, carry):
        _zero_own_rows()
        plsc.subcore_barrier()

        bufs = (rows0, rows1)
        gsems = (gsem0, gsem1)
        ssems = (ssem0, ssem1)

        def _block(p, carry2):
            base = p * U
            gd = [None] * U
            sd = [None] * U
            gd[0] = pltpu.async_copy(
                x_ref.at[i].at[src_v.at[base]], bufs[0], gsems[0])
            for j in range(U):
                gd[j].wait()
                sd[j] = pltpu.async_copy(
                    bufs[j % 2], y_sh.at[dst_v.at[base + j]],
                    ssems[j % 2], add=True)
                if j + 1 < U:
                    if j >= 1:
                        sd[j - 1].wait()
                    gd[j + 1] = pltpu.async_copy(
                        x_ref.at[i].at[src_v.at[base + j + 1]],
                        bufs[(j + 1) % 2], gsems[(j + 1) % 2])
            sd[U - 2].wait()
            sd[U - 1].wait()
            return carry2
        lax.fori_loop(0, CPT // U, _block, 0)
        plsc.subcore_barrier()

        @pl.when(c == 0)
        def _():
            for j in range(RPT // CH):
                pltpu.sync_copy(y_sh.at[pl.ds(s * RPT + j * CH, CH)],
                                y0_ref.at[i, s, j])

        @pl.when(c == 1)
        def _():
            for j in range(RPT // CH):
                pltpu.sync_copy(y_sh.at[pl.ds(s * RPT + j * CH, CH)],
                                y1_ref.at[i, s, j])
        plsc.subcore_barrier()
        return carry
    lax.fori_loop(0, g, _inst, 0)


def _make_sc_agg(with_deg, g):
    yshape = jax.ShapeDtypeStruct((g, NS, RPT // CH, CH, H_GNN), jnp.float32)
    dshape = jax.ShapeDtypeStruct((NS, RPT, H_GNN), jnp.float32)
    out_type = (yshape, yshape, dshape, dshape) if with_deg else (yshape, yshape)
    scratch = [
        pltpu.VMEM((CPT, CH), jnp.int32),
        pltpu.VMEM((CPT, CH), jnp.int32),
        pltpu.VMEM((CH, H_GNN), jnp.float32),
        pltpu.VMEM((CH, H_GNN), jnp.float32),
    ]
    scratch += [pltpu.VMEM_SHARED((N, H_GNN), jnp.float32)]
    scratch += [pltpu.SemaphoreType.DMA] * 9
    mesh = plsc.VectorSubcoreMesh(core_axis_name="c", subcore_axis_name="s")
    return pl.kernel(
        functools.partial(_sc_agg_body, with_deg, g),
        out_type=out_type,
        mesh=mesh,
        scratch_types=scratch,
    )


BN = 2000
NB = N // BN


def _tc_layer1_body(y0, y1, d0, d1, wt, b, z_out):
    y = y0[0] + y1[0]
    deg = d0[:, 0:1] + d1[:, 0:1]
    m = y * (1.0 / jnp.maximum(deg, 1.0))
    z = jnp.dot(m, wt[...], preferred_element_type=jnp.float32) + b[...]
    z_out[0] = jnp.maximum(z, 0.0)


def _tc_layer2_body(y0, y1, d0, d1, wt, b, r_out):
    y = y0[0] + y1[0]
    deg = d0[:, 0:1] + d1[:, 0:1]
    m = y * (1.0 / jnp.maximum(deg, 1.0))
    z = jnp.dot(m, wt[...], preferred_element_type=jnp.float32) + b[...]
    z = jnp.maximum(z, 0.0)
    r_out[...] = (jnp.sum(z, axis=0, keepdims=True) * (1.0 / N))[None, None]


def _tc_layer(emit_z, g):
    in_specs = [
        pl.BlockSpec((1, BN, H_GNN), lambda nb, i: (i, nb, 0)),
        pl.BlockSpec((1, BN, H_GNN), lambda nb, i: (i, nb, 0)),
        pl.BlockSpec((BN, H_GNN), lambda nb, i: (nb, 0)),
        pl.BlockSpec((BN, H_GNN), lambda nb, i: (nb, 0)),
        pl.BlockSpec((H_GNN, H_GNN), lambda nb, i: (0, 0)),
        pl.BlockSpec((1, H_GNN), lambda nb, i: (0, 0)),
    ]
    if emit_z:
        return pl.pallas_call(
            _tc_layer1_body,
            grid=(NB, g),
            in_specs=in_specs,
            out_specs=pl.BlockSpec((1, BN, H_GNN), lambda nb, i: (i, nb, 0)),
            out_shape=jax.ShapeDtypeStruct((g, N, H_GNN), jnp.float32),
        )
    return pl.pallas_call(
        _tc_layer2_body,
        grid=(NB, g),
        in_specs=in_specs,
        out_specs=pl.BlockSpec((1, 1, 1, H_GNN), lambda nb, i: (nb, i, 0, 0)),
        out_shape=jax.ShapeDtypeStruct((NB, g, 1, H_GNN), jnp.float32),
    )


def _gru_heads_body(g_ref, s_ref, t_ref,
                    wihg, whhg, bihg, bhhg,
                    wihs, whhs, bihs, bhhs,
                    wiht, whht, biht, bhht,
                    wfc, bfc, wst, bst, wca, bca,
                    stim_ref, cause_ref):
    H = H_FC

    def gru(seq, wih, whh, bih, bhh):
        h = jnp.zeros((B, H), jnp.float32)
        hs = []
        for t in range(T):
            x = seq[:, t, :]
            gi = jnp.dot(x, wih[...], preferred_element_type=jnp.float32) + bih[...]
            gh = jnp.dot(h, whh[...], preferred_element_type=jnp.float32) + bhh[...]
            r = jax.nn.sigmoid(gi[:, 0:H] + gh[:, 0:H])
            z = jax.nn.sigmoid(gi[:, H:2 * H] + gh[:, H:2 * H])
            n = jnp.tanh(gi[:, 2 * H:3 * H] + r * gh[:, 2 * H:3 * H])
            h = (1.0 - z) * n + z * h
            hs.append(h)
        return hs

    hg = gru(jnp.sum(g_ref[...], axis=0), wihg, whhg, bihg, bhhg)
    hs_ = gru(s_ref[...], wihs, whhs, bihs, bhhs)
    ht = gru(t_ref[...], wiht, whht, biht, bhht)
    for t in range(T):
        cat = jnp.concatenate([hg[t], hs_[t], ht[t]], axis=1)
        hO = jnp.dot(cat, wfc[...], preferred_element_type=jnp.float32) + bfc[...]
        hO = jnp.maximum(hO, 0.0)
        stim_ref[:, t, :] = jnp.dot(hO, wst[...], preferred_element_type=jnp.float32) + bst[...]
        cause_ref[:, t, :] = jnp.dot(hO, wca[...], preferred_element_type=jnp.float32) + bca[...]


_gru_heads = pl.pallas_call(
    _gru_heads_body,
    out_shape=(jax.ShapeDtypeStruct((B, T, OUT_S), jnp.float32),
               jax.ShapeDtypeStruct((B, T, OUT_C), jnp.float32)),
)


def kernel(node_feats, edge_index, bSensor, bTarget, bArea,
           W_gnn1, b_gnn1, W_gnn3, b_gnn3,
           W_ih_G, W_hh_G, b_ih_G, b_hh_G,
           W_ih_S, W_hh_S, b_ih_S, b_hh_S,
           W_ih_T, W_hh_T, b_ih_T, b_hh_T,
           W_fc1, b_fc1, W_stim, b_stim, W_cause, b_cause):
    src_rows = edge_index[0].reshape(NW, CPT, CH)
    dst_rows = edge_index[1].reshape(NW, CPT, CH)

    GQ = G // 4
    x1 = node_feats.reshape(G, N, D_IN)
    wt1, bb1 = W_gnn1.T, b_gnn1.reshape(1, -1)
    wt3, bb3 = W_gnn3.T, b_gnn3.reshape(1, -1)

    sc_l = _make_sc_agg(False, GQ)
    ya = [None] * 4
    ya[0] = _make_sc_agg(True, GQ)(x1[0:GQ], src_rows, dst_rows)
    deg0 = ya[0][2].reshape(N, H_GNN)
    deg1 = ya[0][3].reshape(N, H_GNN)
    for q in range(1, 4):
        ya[q] = sc_l(x1[q * GQ:(q + 1) * GQ], src_rows, dst_rows)
    z1 = [_tc_layer(True, GQ)(ya[q][0].reshape(GQ, N, H_GNN),
                              ya[q][1].reshape(GQ, N, H_GNN),
                              deg0, deg1, wt1, bb1) for q in range(4)]
    yb = [sc_l(z1[q], src_rows, dst_rows) for q in range(4)]
    rps = [_tc_layer(False, GQ)(yb[q][0].reshape(GQ, N, H_GNN),
                                yb[q][1].reshape(GQ, N, H_GNN),
                                deg0, deg1, wt3, bb3) for q in range(4)]
    rp = jnp.concatenate(rps, axis=1)

    stim4, cause4 = _gru_heads(
        rp.reshape(NB, B, T, H_RNN), bSensor, bTarget,
        W_ih_G.T, W_hh_G.T, b_ih_G.reshape(1, -1), b_hh_G.reshape(1, -1),
        W_ih_S.T, W_hh_S.T, b_ih_S.reshape(1, -1), b_hh_S.reshape(1, -1),
        W_ih_T.T, W_hh_T.T, b_ih_T.reshape(1, -1), b_hh_T.reshape(1, -1),
        W_fc1.T, b_fc1.reshape(1, -1),
        W_stim.T, b_stim.reshape(1, -1),
        W_cause.T, b_cause.reshape(1, -1))
    return (stim4.reshape(B * T, OUT_S), cause4.reshape(B * T, OUT_C))
